# baseline (speedup 1.0000x reference)
import functools

import jax
import jax.numpy as jnp
from jax import lax
from jax.experimental import pallas as pl
from jax.experimental.pallas import tpu as pltpu
from jax.experimental.pallas import tpu_sc as plsc

_B, _L, _V, _D = 16384, 200, 1000000, 32
_H = 150
_NC, _NS = 2, 16
_NW = _NC * _NS
_EPW = _B // _NW
_CH = 64
_BV = 2048
_NBLK = (_V + _BV - 1) // _BV
_L0, _L1 = 128, 72


def _repack_table_tc(tabT):
  def body(i_ref, o_ref):
    for c in range(4):
      o_ref[:, 32 * c:32 * (c + 1)] = jnp.transpose(
          i_ref[:, 512 * c:512 * (c + 1)])

  return pl.pallas_call(
      body,
      grid=(_NBLK,),
      in_specs=[pl.BlockSpec((_D, _BV), lambda i: (0, i))],
      out_specs=pl.BlockSpec((_BV // 4, 128), lambda i: (i, 0)),
      out_shape=jax.ShapeDtypeStruct((_NBLK * (_BV // 4), 128), jnp.float32),
  )(tabT)


def _repack_x_tc(xT):
  def body(i_ref, o_ref):
    t = jnp.transpose(i_ref[...])
    phi = (t & -2048) | ((t & 511) << 2) | ((t >> 9) & 3)
    o_ref[...] = phi

  return pl.pallas_call(
      body,
      grid=(2, _B // 128),
      in_specs=[pl.BlockSpec((128, 128), lambda h, i: (h, i))],
      out_specs=pl.BlockSpec(
          (128, 128), lambda h, i: (h * (_B // 128) + i, 0)),
      out_shape=jax.ShapeDtypeStruct((2 * _B, 128), jnp.int32),
  )(xT)


def _pool_sc(xfp, tab_lin):
  mesh = plsc.VectorSubcoreMesh(core_axis_name="c", subcore_axis_name="s")

  @functools.partial(
      pl.kernel,
      mesh=mesh,
      compiler_params=pltpu.CompilerParams(use_tc_tiling_on_sc=False),
      out_type=jax.ShapeDtypeStruct((_B, _D), jnp.float32),
      scratch_types=[
          pltpu.VMEM((2, _CH, 128), jnp.int32),
          pltpu.VMEM((2, _L0, _D), jnp.float32),
          pltpu.VMEM((2, _L1, _D), jnp.float32),
          pltpu.VMEM((_CH, _D), jnp.float32),
          pltpu.SemaphoreType.DMA,
          pltpu.SemaphoreType.DMA,
      ],
  )
  def body(x_hbm, tab_hbm, out_hbm, idx_v, rows_a, rows_b, pool_v,
           sem_a, sem_b):
    wid = lax.axis_index("s") * _NC + lax.axis_index("c")
    base = wid * _EPW

    def fire(j, buf, sem):
      pltpu.async_copy(tab_hbm.at[idx_v.at[0, j]], rows_a.at[buf], sem)
      pltpu.async_copy(tab_hbm.at[idx_v.at[1, j, pl.ds(0, _L1)]],
                       rows_b.at[buf], sem)

    def drain(buf, sem):
      pltpu.make_async_copy(
          tab_hbm.at[idx_v.at[0, 0]], rows_a.at[buf], sem).wait()
      pltpu.make_async_copy(
          tab_hbm.at[idx_v.at[1, 0, pl.ds(0, _L1)]], rows_b.at[buf],
          sem).wait()

    def accum(j, buf):
      def row2_body(r, accs):
        a0, a1 = accs
        a0 = a0 + rows_a[buf, r, pl.ds(0, 16)]
        a0 = a0 + rows_b[buf, r, pl.ds(0, 16)]
        a1 = a1 + rows_a[buf, r, pl.ds(16, 16)]
        a1 = a1 + rows_b[buf, r, pl.ds(16, 16)]
        return (a0, a1)

      def row1_body(r, accs):
        a0, a1 = accs
        a0 = a0 + rows_a[buf, r, pl.ds(0, 16)]
        a1 = a1 + rows_a[buf, r, pl.ds(16, 16)]
        return (a0, a1)

      z = jnp.zeros((16,), jnp.float32)
      accs = lax.fori_loop(0, _L1, row2_body, (z, z))
      a0, a1 = lax.fori_loop(_L1, _L0, row1_body, accs)
      pool_v[j, pl.ds(0, 16)] = a0 * (1.0 / _L)
      pool_v[j, pl.ds(16, 16)] = a1 * (1.0 / _L)

    def chunk_body(ci, carry):
      cbase = base + ci * _CH
      pltpu.sync_copy(x_hbm.at[0, pl.ds(cbase, _CH)], idx_v.at[0])
      pltpu.sync_copy(x_hbm.at[1, pl.ds(cbase, _CH)], idx_v.at[1])
      fire(0, 0, sem_a)

      def pair_body(p, carry2):
        j0 = 2 * p
        fire(j0 + 1, 1, sem_b)
        drain(0, sem_a)
        accum(j0, 0)

        @pl.when(j0 + 2 < _CH)
        def _():
          fire(j0 + 2, 0, sem_a)

        drain(1, sem_b)
        accum(j0 + 1, 1)
        return carry2

      lax.fori_loop(0, _CH // 2, pair_body, 0)
      pltpu.sync_copy(pool_v, out_hbm.at[pl.ds(cbase, _CH)])
      return carry

    lax.fori_loop(0, _EPW // _CH, chunk_body, 0)

  return body(xfp, tab_lin)


def _mlp_tc(pooled, W1, b1, W2, b2, W3, b3):
  bb = 2048

  def body(p_ref, w1_ref, b1_ref, w2_ref, b2_ref, w3_ref, b3_ref, o_ref):
    h = jnp.dot(p_ref[...], w1_ref[...], preferred_element_type=jnp.float32)
    h = jnp.maximum(h + b1_ref[...], 0.0)
    h = jnp.dot(h, w2_ref[...], preferred_element_type=jnp.float32)
    h = jnp.maximum(h + b2_ref[...], 0.0)
    o_ref[...] = (
        jnp.dot(h, w3_ref[...], preferred_element_type=jnp.float32)
        + b3_ref[...]
    )

  return pl.pallas_call(
      body,
      grid=(_B // bb,),
      in_specs=[
          pl.BlockSpec((bb, _D), lambda i: (i, 0)),
          pl.BlockSpec((_D, _H), lambda i: (0, 0)),
          pl.BlockSpec((1, _H), lambda i: (0, 0)),
          pl.BlockSpec((_H, _H), lambda i: (0, 0)),
          pl.BlockSpec((1, _H), lambda i: (0, 0)),
          pl.BlockSpec((_H, 1), lambda i: (0, 0)),
          pl.BlockSpec((1, 1), lambda i: (0, 0)),
      ],
      out_specs=pl.BlockSpec((bb, 1), lambda i: (i, 0)),
      out_shape=jax.ShapeDtypeStruct((_B, 1), jnp.float32),
  )(pooled, W1, b1.reshape(1, _H), W2, b2.reshape(1, _H), W3,
    b3.reshape(1, 1))


@jax.jit
def _run(x, table, W1, b1, W2, b2, W3, b3):
  packed = _repack_table_tc(table.T)
  tab_lin = packed.reshape(_NBLK * _BV, _D)
  xfp = _repack_x_tc(x.astype(jnp.int32).T).reshape(2, _B, 128)
  pooled = _pool_sc(xfp, tab_lin)
  return _mlp_tc(pooled, W1, b1, W2, b2, W3, b3)


def kernel(x, table, W1, b1, W2, b2, W3, b3):
  return _run(x, table, W1, b1, W2, b2, W3, b3)
